# baseline (speedup 1.0000x reference)
import functools

import jax
import jax.numpy as jnp
from jax import lax
from jax.experimental import pallas as pl
from jax.experimental.pallas import tpu as pltpu
from jax.experimental.pallas import tpu_sc as plsc

NROWS = 128
NCOLS = 32768
K = 64
NC, NS, L = 2, 16, 16
NW = NC * NS
ROWS_PER_W = NROWS // NW
NVREG = NCOLS // L
NBINS = 256


def _body(x_hbm, out_hbm, row_v, keys_v, idxs_v, hist_v, dkey_v, didx_v,
          ostage_v, sem):
    wid = lax.axis_index("s") * NC + lax.axis_index("c")
    row0 = wid * ROWS_PER_W
    lanes = lax.iota(jnp.int32, L)
    ones = jnp.ones((L,), jnp.int32)
    zeros = jnp.zeros((L,), jnp.int32)
    tmask = lanes >= 0

    pltpu.make_async_copy(x_hbm.at[row0], row_v, sem).start()

    def zero_hist():
        def zb(i, _):
            hist_v[pl.ds(i * L, L)] = zeros
            return 0
        lax.fori_loop(0, (NBINS * L) // L, zb, 0)

    def pass1():
        def p1(j, _):
            x = row_v[pl.ds(j * L, L)]
            k = lax.bitcast_convert_type(x, jnp.int32)
            ikey = jnp.where(k < 0, k ^ jnp.int32(0x7FFFFFFF), k)
            keys_v[pl.ds(j * L, L)] = ikey
            idxs_v[pl.ds(j * L, L)] = j * L + lanes
            digit = ((ikey >> 24) & 0xFF) ^ 0x80
            plsc.addupdate_scatter(hist_v, [digit + lanes * NBINS], ones,
                                   mask=tmask)
            return 0
        lax.fori_loop(0, NVREG, p1, 0)

    def hist_pass(M, shift):
        zero_hist()
        def hb(j, _):
            base = j * L
            kv = keys_v[pl.ds(base, L)]
            valid = (base + lanes) < M
            digit = (kv >> shift) & 0xFF
            plsc.addupdate_scatter(hist_v, [digit + lanes * NBINS], ones,
                                   mask=valid)
            return 0
        lax.fori_loop(0, (M + L - 1) // L, hb, 0)

    def find_bin(kneed):
        rcs = []
        bss = []
        for v in range(L):
            def lr(l, acc):
                return acc + hist_v[pl.ds(l * NBINS + v * L, L)]
            t_v = lax.fori_loop(0, L, lr, zeros)
            rc_v = lax.rev(plsc.cumsum(lax.rev(t_v, (0,))), (0,))
            rcs.append(rc_v)
            bss.append(jnp.max(rc_v))
        b = jnp.int32(-1)
        found = jnp.bool_(False)
        sab = jnp.int32(0)
        for v in range(L - 1, -1, -1):
            need = kneed - sab
            cnt = jnp.sum((rcs[v] >= need).astype(jnp.int32))
            hit = cnt > 0
            b = jnp.where(found | (~hit), b, v * L + cnt - 1)
            found = found | hit
            sab = sab + bss[v]
        return b

    def split(M, shift, xor, b, doff):
        def sb(j, carry):
            doff, boff = carry
            base = j * L
            kv = keys_v[pl.ds(base, L)]
            iv = idxs_v[pl.ds(base, L)]
            valid = (base + lanes) < M
            digit = ((kv >> shift) & 0xFF) ^ xor
            m_def = (digit > b) & valid
            m_bnd = (digit == b) & valid
            c_def = plsc.cumsum(m_def.astype(jnp.int32))
            dd = doff + c_def - 1
            plsc.store_scatter(dkey_v, [dd], kv, mask=m_def)
            plsc.store_scatter(didx_v, [dd], iv, mask=m_def)
            c_bnd = plsc.cumsum(m_bnd.astype(jnp.int32))
            db = boff + c_bnd - 1
            plsc.store_scatter(keys_v, [db], kv, mask=m_bnd)
            plsc.store_scatter(idxs_v, [db], iv, mask=m_bnd)
            return doff + jnp.max(c_def), boff + jnp.max(c_bnd)
        return lax.fori_loop(0, (M + L - 1) // L, sb,
                             (doff, jnp.int32(0)))

    def row_body(r, _):
        row = row0 + r
        pltpu.make_async_copy(x_hbm.at[row], row_v, sem).wait()
        zero_hist()
        pass1()

        @pl.when(r < ROWS_PER_W - 1)
        def _():
            pltpu.make_async_copy(x_hbm.at[row + 1], row_v, sem).start()

        kneed = jnp.int32(K)
        doff = jnp.int32(0)
        M = jnp.int32(NCOLS)
        for lvl, (shift, xor) in enumerate(
                [(24, 0x80), (16, 0), (8, 0), (0, 0)]):
            if lvl > 0:
                hist_pass(M, shift)
            b = find_bin(kneed)
            doff2, M = split(M, shift, xor, b, doff)
            kneed = kneed - (doff2 - doff)
            doff = doff2

        def ap(j, _):
            pos = j * L + lanes
            m = pos < kneed
            kv = keys_v[pl.ds(j * L, L)]
            iv = idxs_v[pl.ds(j * L, L)]
            plsc.store_scatter(dkey_v, [doff + pos], kv, mask=m)
            plsc.store_scatter(didx_v, [doff + pos], iv, mask=m)
            return 0
        lax.fori_loop(0, (kneed + L - 1) // L, ap, 0)

        Ks = [dkey_v[pl.ds(a * L, L)] for a in range(K // L)]
        Is = [didx_v[pl.ds(a * L, L)] for a in range(K // L)]
        lane0 = lanes == 0
        def rk(c, _):
            csplat = jnp.full((L,), c, jnp.int32)
            kc = plsc.load_gather(dkey_v, [csplat])
            ic = plsc.load_gather(didx_v, [csplat])
            rank = zeros
            for a in range(K // L):
                m = (Ks[a] > kc) | ((Ks[a] == kc) & (Is[a] < ic))
                rank = rank + plsc.all_reduce_population_count(m)
            plsc.store_scatter(ostage_v, [rank], ic, mask=lane0)
            return 0
        lax.fori_loop(0, K, rk, 0)

        pltpu.make_async_copy(ostage_v, out_hbm.at[row], sem).start()
        pltpu.make_async_copy(ostage_v, out_hbm.at[row], sem).wait()
        return 0

    lax.fori_loop(0, ROWS_PER_W, row_body, 0)


@jax.jit
def kernel(input_tensor):
    mesh = plsc.VectorSubcoreMesh(core_axis_name="c", subcore_axis_name="s",
                                  num_cores=NC, num_subcores=NS)
    f = pl.kernel(
        _body,
        out_type=jax.ShapeDtypeStruct((NROWS, K), jnp.int32),
        mesh=mesh,
        scratch_types=[
            pltpu.VMEM((NCOLS,), jnp.float32),
            pltpu.VMEM((NCOLS,), jnp.int32),
            pltpu.VMEM((NCOLS,), jnp.int32),
            pltpu.VMEM((NBINS * L,), jnp.int32),
            pltpu.VMEM((K,), jnp.int32),
            pltpu.VMEM((K,), jnp.int32),
            pltpu.VMEM((K,), jnp.int32),
            pltpu.SemaphoreType.DMA,
        ],
        compiler_params=pltpu.CompilerParams(
            needs_layout_passes=False,
            use_tc_tiling_on_sc=False,
        ),
    )
    return f(input_tensor)


# speedup vs baseline: 1.0822x; 1.0822x over previous
import functools

import jax
import jax.numpy as jnp
from jax import lax
from jax.experimental import pallas as pl
from jax.experimental.pallas import tpu as pltpu
from jax.experimental.pallas import tpu_sc as plsc

NROWS = 128
NCOLS = 32768
K = 64
NC, NS, L = 2, 16, 16
NW = NC * NS
ROWS_PER_W = NROWS // NW
NVREG = NCOLS // L
NBINS = 256


def _body(x_hbm, out_hbm, row_v, keys_v, idxs_v, hist_v, dkey_v, didx_v,
          ostage_v, sem):
    wid = lax.axis_index("s") * NC + lax.axis_index("c")
    row0 = wid * ROWS_PER_W
    lanes = lax.iota(jnp.int32, L)
    ones = jnp.ones((L,), jnp.int32)
    zeros = jnp.zeros((L,), jnp.int32)
    tmask = lanes >= 0

    pltpu.make_async_copy(x_hbm.at[row0], row_v, sem).start()

    lanebins = lanes * NBINS

    def zero_hist():
        def zb(i, _):
            hist_v[pl.ds(i * L, L)] = zeros
            return 0
        lax.fori_loop(0, (NBINS * L) // L, zb, 0, unroll=8)

    def pass1():
        def p1(j, _):
            x = row_v[pl.ds(j * L, L)]
            k = lax.bitcast_convert_type(x, jnp.int32)
            ikey = jnp.where(k < 0, k ^ jnp.int32(0x7FFFFFFF), k)
            keys_v[pl.ds(j * L, L)] = ikey
            idxs_v[pl.ds(j * L, L)] = j * L + lanes
            digit = ((ikey >> 24) & 0xFF) ^ 0x80
            plsc.addupdate_scatter(hist_v, [digit + lanebins], ones,
                                   mask=tmask)
            return 0
        lax.fori_loop(0, NVREG, p1, 0, unroll=8)

    def hist_pass(M, shift):
        zero_hist()
        def hb(j, _):
            base = j * L
            kv = keys_v[pl.ds(base, L)]
            valid = (base + lanes) < M
            digit = (kv >> shift) & 0xFF
            plsc.addupdate_scatter(hist_v, [digit + lanes * NBINS], ones,
                                   mask=valid)
            return 0
        lax.fori_loop(0, (M + L - 1) // L, hb, 0)

    def find_bin(kneed):
        rcs = []
        bss = []
        for v in range(L):
            def lr(l, acc):
                return acc + hist_v[pl.ds(l * NBINS + v * L, L)]
            t_v = lax.fori_loop(0, L, lr, zeros, unroll=16)
            rc_v = lax.rev(plsc.cumsum(lax.rev(t_v, (0,))), (0,))
            rcs.append(rc_v)
            bss.append(jnp.max(rc_v))
        b = jnp.int32(-1)
        found = jnp.bool_(False)
        sab = jnp.int32(0)
        for v in range(L - 1, -1, -1):
            need = kneed - sab
            cnt = jnp.sum((rcs[v] >= need).astype(jnp.int32))
            hit = cnt > 0
            b = jnp.where(found | (~hit), b, v * L + cnt - 1)
            found = found | hit
            sab = sab + bss[v]
        return b

    def split_step(base, valid, b, shift, xor, doff, boff):
        kv = keys_v[pl.ds(base, L)]
        iv = idxs_v[pl.ds(base, L)]
        digit = ((kv >> shift) & 0xFF) ^ xor
        m_def = (digit > b) & valid
        m_bnd = (digit == b) & valid
        c_def = plsc.cumsum(m_def.astype(jnp.int32))
        dd = doff + c_def - 1
        plsc.store_scatter(dkey_v, [dd], kv, mask=m_def)
        plsc.store_scatter(didx_v, [dd], iv, mask=m_def)
        c_bnd = plsc.cumsum(m_bnd.astype(jnp.int32))
        db = boff + c_bnd - 1
        plsc.store_scatter(keys_v, [db], kv, mask=m_bnd)
        plsc.store_scatter(idxs_v, [db], iv, mask=m_bnd)
        doff = doff + plsc.all_reduce_population_count(m_def)
        boff = boff + plsc.all_reduce_population_count(m_bnd)
        return doff, boff

    def split1(b, doff):
        def sb(j, carry):
            return split_step(j * L, tmask, b, 24, 0x80, *carry)
        return lax.fori_loop(0, NVREG, sb, (doff, zeros), unroll=4)

    def split(M, shift, b, doff):
        def sb(j, carry):
            base = j * L
            return split_step(base, (base + lanes) < M, b, shift, 0, *carry)
        return lax.fori_loop(0, (M + L - 1) // L, sb, (doff, zeros))

    def row_body(r, _):
        row = row0 + r
        pltpu.make_async_copy(x_hbm.at[row], row_v, sem).wait()
        zero_hist()
        pass1()

        @pl.when(r < ROWS_PER_W - 1)
        def _():
            pltpu.make_async_copy(x_hbm.at[row + 1], row_v, sem).start()

        kneed = jnp.int32(K)
        doff = zeros
        M = jnp.int32(NCOLS)
        for lvl, shift in enumerate([24, 16, 8, 0]):
            if lvl > 0:
                hist_pass(M, shift)
            b = find_bin(kneed)
            if lvl == 0:
                doff, boff = split1(b, doff)
            else:
                doff, boff = split(M, shift, b, doff)
            kneed = K - jnp.max(doff)
            M = jnp.max(boff)
        doff = jnp.max(doff)

        def ap(j, _):
            pos = j * L + lanes
            m = pos < kneed
            kv = keys_v[pl.ds(j * L, L)]
            iv = idxs_v[pl.ds(j * L, L)]
            plsc.store_scatter(dkey_v, [doff + pos], kv, mask=m)
            plsc.store_scatter(didx_v, [doff + pos], iv, mask=m)
            return 0
        lax.fori_loop(0, (kneed + L - 1) // L, ap, 0)

        Ks = [dkey_v[pl.ds(a * L, L)] for a in range(K // L)]
        Is = [didx_v[pl.ds(a * L, L)] for a in range(K // L)]
        lane0 = lanes == 0
        def rk(c, _):
            csplat = jnp.full((L,), c, jnp.int32)
            kc = plsc.load_gather(dkey_v, [csplat])
            ic = plsc.load_gather(didx_v, [csplat])
            rank = zeros
            for a in range(K // L):
                m = (Ks[a] > kc) | ((Ks[a] == kc) & (Is[a] < ic))
                rank = rank + plsc.all_reduce_population_count(m)
            plsc.store_scatter(ostage_v, [rank], ic, mask=lane0)
            return 0
        lax.fori_loop(0, K, rk, 0)

        pltpu.make_async_copy(ostage_v, out_hbm.at[row], sem).start()
        pltpu.make_async_copy(ostage_v, out_hbm.at[row], sem).wait()
        return 0

    lax.fori_loop(0, ROWS_PER_W, row_body, 0)


@jax.jit
def kernel(input_tensor):
    mesh = plsc.VectorSubcoreMesh(core_axis_name="c", subcore_axis_name="s",
                                  num_cores=NC, num_subcores=NS)
    f = pl.kernel(
        _body,
        out_type=jax.ShapeDtypeStruct((NROWS, K), jnp.int32),
        mesh=mesh,
        scratch_types=[
            pltpu.VMEM((NCOLS,), jnp.float32),
            pltpu.VMEM((NCOLS,), jnp.int32),
            pltpu.VMEM((NCOLS,), jnp.int32),
            pltpu.VMEM((NBINS * L,), jnp.int32),
            pltpu.VMEM((K,), jnp.int32),
            pltpu.VMEM((K,), jnp.int32),
            pltpu.VMEM((K,), jnp.int32),
            pltpu.SemaphoreType.DMA,
        ],
        compiler_params=pltpu.CompilerParams(
            needs_layout_passes=False,
            use_tc_tiling_on_sc=False,
        ),
    )
    return f(input_tensor)


# speedup vs baseline: 2.1705x; 2.0056x over previous
import jax
import jax.numpy as jnp
from jax import lax
from jax.experimental import pallas as pl
from jax.experimental.pallas import tpu as pltpu
from jax.experimental.pallas import tpu_sc as plsc

NROWS = 128
NCOLS = 32768
K = 64
NC, NS, L = 2, 16, 16
NW = NC * NS
ROWS_PER_W = NROWS // NW
NVREG = NCOLS // L
NBINS = 256
HSTRIDE = NBINS + 1
HSIZE = HSTRIDE * L
NHIST = 4
NCHUNK = 4
CWORDS = NCOLS // NCHUNK
CVREG = CWORDS // L


def _body(x_hbm, out_hbm, row_v, keys_v, idxs_v, hist_v, dkey_v, didx_v,
          ostage_v, sems, sem_out):
    wid = lax.axis_index("s") * NC + lax.axis_index("c")
    row0 = wid * ROWS_PER_W
    lanes = lax.iota(jnp.int32, L)
    ones = jnp.ones((L,), jnp.int32)
    zeros = jnp.zeros((L,), jnp.int32)
    tmask = lanes >= 0
    lane0 = lanes == 0
    lanebins = lanes * HSTRIDE


    def zero_hist(ncopies):
        @plsc.parallel_loop(0, (HSTRIDE * ncopies * L) // L, unroll=4)
        def _zb(i):
            hist_v[pl.ds(i * L, L)] = zeros

    def pass1_chunk(c0):
        @plsc.parallel_loop(0, CVREG // NHIST, unroll=2)
        def _p1(j):
            for t in range(NHIST):
                base = c0 + (j * NHIST + t) * L
                k = row_v[pl.ds(base, L)]
                ikey = jnp.where(k < 0, k ^ jnp.int32(0x7FFFFFFF), k)
                keys_v[pl.ds(base, L)] = ikey
                digit = ((ikey >> 24) & 0xFF) ^ 0x80
                plsc.addupdate_scatter(
                    hist_v, [digit + (lanebins + t * HSIZE)], ones,
                    mask=tmask)

    def hist_pass(M, shift):
        zero_hist(1)
        def hb(j, _):
            base = j * L
            kv = row_v[pl.ds(base, L)]
            valid = (base + lanes) < M
            digit = (kv >> shift) & 0xFF
            plsc.addupdate_scatter(hist_v, [digit + lanebins], ones,
                                   mask=valid)
            return 0
        lax.fori_loop(0, (M + L - 1) // L, hb, 0)

    def find_bin(kneed, ncopies):
        b = jnp.int32(-1)
        found = jnp.bool_(False)
        sab = jnp.int32(0)
        for v in range(L - 1, -1, -1):
            def lr(l, acc):
                a = acc
                for c in range(ncopies):
                    a = a + hist_v[pl.ds(l * HSTRIDE + c * HSIZE + v * L, L)]
                return a
            t_v = lax.fori_loop(0, L, lr, zeros, unroll=4)
            rc_v = lax.rev(plsc.cumsum(lax.rev(t_v, (0,))), (0,))
            need = kneed - sab
            cnt = jnp.sum((rc_v >= need).astype(jnp.int32))
            hit = cnt > 0
            b = jnp.where(found | (~hit), b, v * L + cnt - 1)
            found = found | hit
            sab = sab + jnp.max(rc_v)
        return b

    def split1(b, doff):
        @plsc.parallel_loop(0, NVREG, unroll=2, carry=(doff, zeros))
        def _sb(j, carry):
            doff, boff = carry
            base = j * L
            kv = keys_v[pl.ds(base, L)]
            iv = base + lanes
            digit = ((kv >> 24) & 0xFF) ^ 0x80
            m_def = digit > b
            m_bnd = digit == b
            c_def = plsc.cumsum(m_def.astype(jnp.int32))
            dd = doff + c_def - 1
            plsc.store_scatter(dkey_v, [dd], kv, mask=m_def)
            plsc.store_scatter(didx_v, [dd], iv, mask=m_def)
            c_bnd = plsc.cumsum(m_bnd.astype(jnp.int32))
            db = boff + c_bnd - 1
            plsc.store_scatter(row_v, [db], kv, mask=m_bnd)
            plsc.store_scatter(idxs_v, [db], iv, mask=m_bnd)
            doff = doff + plsc.all_reduce_population_count(m_def)
            boff = boff + plsc.all_reduce_population_count(m_bnd)
            return doff, boff
        return _sb

    def split(M, shift, b, doff):
        def sb(j, carry):
            doff, boff = carry
            base = j * L
            kv = row_v[pl.ds(base, L)]
            iv = idxs_v[pl.ds(base, L)]
            valid = (base + lanes) < M
            digit = (kv >> shift) & 0xFF
            m_def = (digit > b) & valid
            m_bnd = (digit == b) & valid
            c_def = plsc.cumsum(m_def.astype(jnp.int32))
            dd = doff + c_def - 1
            plsc.store_scatter(dkey_v, [dd], kv, mask=m_def)
            plsc.store_scatter(didx_v, [dd], iv, mask=m_def)
            c_bnd = plsc.cumsum(m_bnd.astype(jnp.int32))
            db = boff + c_bnd - 1
            plsc.store_scatter(row_v, [db], kv, mask=m_bnd)
            plsc.store_scatter(idxs_v, [db], iv, mask=m_bnd)
            doff = doff + plsc.all_reduce_population_count(m_def)
            boff = boff + plsc.all_reduce_population_count(m_bnd)
            return doff, boff
        return lax.fori_loop(0, (M + L - 1) // L, sb, (doff, zeros))

    def chunk_copy(row, c):
        return pltpu.make_async_copy(
            x_hbm.at[row, pl.ds(c * CWORDS, CWORDS)],
            row_v.at[pl.ds(c * CWORDS, CWORDS)], sems[c])

    def row_body(r, _):
        row = row0 + r
        for c in range(NCHUNK):
            chunk_copy(row, c).start()
        zero_hist(NHIST)
        for c in range(NCHUNK):
            chunk_copy(row, c).wait()
            pass1_chunk(c * CWORDS)

        b = find_bin(jnp.int32(K), NHIST)
        doff, boff = split1(b, zeros)

        kneed = K - jnp.max(doff)
        M = jnp.max(boff)
        for shift in (16, 8, 0):
            hist_pass(M, shift)
            b = find_bin(kneed, 1)
            doff, boff = split(M, shift, b, doff)
            kneed = K - jnp.max(doff)
            M = jnp.max(boff)
        doffs = jnp.max(doff)

        def ap(j, _):
            pos = j * L + lanes
            m = pos < kneed
            kv = row_v[pl.ds(j * L, L)]
            iv = idxs_v[pl.ds(j * L, L)]
            plsc.store_scatter(dkey_v, [doffs + pos], kv, mask=m)
            plsc.store_scatter(didx_v, [doffs + pos], iv, mask=m)
            return 0
        lax.fori_loop(0, (kneed + L - 1) // L, ap, 0)

        Ks = [dkey_v[pl.ds(a * L, L)] for a in range(K // L)]
        Is = [didx_v[pl.ds(a * L, L)] for a in range(K // L)]
        rsplat = jnp.full((L,), r, jnp.int32)

        @plsc.parallel_loop(0, K, unroll=2)
        def _rk(c):
            csplat = jnp.full((L,), c, jnp.int32)
            kc = plsc.load_gather(dkey_v, [csplat])
            ic = plsc.load_gather(didx_v, [csplat])
            rank = zeros
            for a in range(K // L):
                m = (Ks[a] > kc) | ((Ks[a] == kc) & (Is[a] < ic))
                rank = rank + plsc.all_reduce_population_count(m)
            plsc.store_scatter(ostage_v, [rsplat, rank], ic, mask=lane0)
        return 0

    lax.fori_loop(0, ROWS_PER_W, row_body, 0)
    pltpu.make_async_copy(ostage_v, out_hbm.at[pl.ds(row0, ROWS_PER_W)],
                          sem_out).start()
    pltpu.make_async_copy(ostage_v, out_hbm.at[pl.ds(row0, ROWS_PER_W)],
                          sem_out).wait()


@jax.jit
def kernel(input_tensor):
    x_i32 = lax.bitcast_convert_type(input_tensor, jnp.int32)
    mesh = plsc.VectorSubcoreMesh(core_axis_name="c", subcore_axis_name="s",
                                  num_cores=NC, num_subcores=NS)
    f = pl.kernel(
        _body,
        out_type=jax.ShapeDtypeStruct((NROWS, K), jnp.int32),
        mesh=mesh,
        scratch_types=[
            pltpu.VMEM((NCOLS,), jnp.int32),
            pltpu.VMEM((NCOLS,), jnp.int32),
            pltpu.VMEM((NCOLS,), jnp.int32),
            pltpu.VMEM((HSIZE * NHIST,), jnp.int32),
            pltpu.VMEM((K,), jnp.int32),
            pltpu.VMEM((K,), jnp.int32),
            pltpu.VMEM((ROWS_PER_W, K), jnp.int32),
            [pltpu.SemaphoreType.DMA] * NCHUNK,
            pltpu.SemaphoreType.DMA,
        ],
        compiler_params=pltpu.CompilerParams(
            needs_layout_passes=False,
            use_tc_tiling_on_sc=False,
        ),
    )
    return f(x_i32)


# speedup vs baseline: 2.4893x; 1.1469x over previous
import jax
import jax.numpy as jnp
from jax import lax
from jax.experimental import pallas as pl
from jax.experimental.pallas import tpu as pltpu
from jax.experimental.pallas import tpu_sc as plsc

NROWS = 128
NCOLS = 32768
K = 64
NC, NS, L = 2, 16, 16
NW = NC * NS
ROWS_PER_W = NROWS // NW
NVREG = NCOLS // L
NBINS = 256
HSTRIDE = NBINS + 1
HSIZE = HSTRIDE * L
NHIST = 4
NCHUNK = 4
CWORDS = NCOLS // NCHUNK
CVREG = CWORDS // L


def _body(x_hbm, out_hbm, row_v, keys_v, idxs_v, hist_v, didx_v,
          ostage_v, sems, sem_out):
    wid = lax.axis_index("s") * NC + lax.axis_index("c")
    row0 = wid * ROWS_PER_W
    lanes = lax.iota(jnp.int32, L)
    ones = jnp.ones((L,), jnp.int32)
    zeros = jnp.zeros((L,), jnp.int32)
    tmask = lanes >= 0
    lane0 = lanes == 0
    lanebins = lanes * HSTRIDE

    def zero_hist(ncopies):
        @plsc.parallel_loop(0, (HSTRIDE * ncopies * L) // L, unroll=4)
        def _zb(i):
            hist_v[pl.ds(i * L, L)] = zeros

    def pass1_chunk(c0):
        @plsc.parallel_loop(0, CVREG // NHIST, unroll=2)
        def _p1(j):
            for t in range(NHIST):
                base = c0 + (j * NHIST + t) * L
                k = row_v[pl.ds(base, L)]
                ikey = jnp.where(k < 0, k ^ jnp.int32(0x7FFFFFFF), k)
                keys_v[pl.ds(base, L)] = ikey
                digit = ((ikey >> 24) & 0xFF) ^ 0x80
                plsc.addupdate_scatter(
                    hist_v, [digit + (lanebins + t * HSIZE)], ones,
                    mask=tmask)

    def hist_pass(M, shift, hi):
        zero_hist(1)
        def hb(j, _):
            base = j * L
            valid = (base + lanes) < M
            iv = idxs_v[pl.ds(base, L)]
            kv = plsc.load_gather(keys_v, [iv], mask=valid)
            digit = jnp.where(kv > hi, 255, (kv >> shift) & 0xFF)
            plsc.addupdate_scatter(hist_v, [digit + lanebins], ones,
                                   mask=valid)
            return 0
        lax.fori_loop(0, (M + L - 1) // L, hb, 0)

    def find_bin(ncopies):
        b = jnp.int32(-1)
        found = jnp.bool_(False)
        sab = jnp.int32(0)
        for v in range(L - 1, -1, -1):
            def lr(l, acc):
                a = acc
                for c in range(ncopies):
                    a = a + hist_v[pl.ds(l * HSTRIDE + c * HSIZE + v * L, L)]
                return a
            t_v = lax.fori_loop(0, L, lr, zeros, unroll=4)
            rc_v = lax.rev(plsc.cumsum(lax.rev(t_v, (0,))), (0,))
            need = K - sab
            cnt = jnp.sum((rc_v >= need).astype(jnp.int32))
            hit = cnt > 0
            b = jnp.where(found | (~hit), b, v * L + cnt - 1)
            found = found | hit
            sab = sab + jnp.max(rc_v)
        return b

    def split1(tp):
        @plsc.parallel_loop(0, NVREG, unroll=4, carry=zeros)
        def _sb(j, boff):
            base = j * L
            kv = keys_v[pl.ds(base, L)]
            m = kv >= tp
            c = plsc.cumsum(m.astype(jnp.int32))
            plsc.store_scatter(idxs_v, [boff + c - 1], base + lanes, mask=m)
            return boff + plsc.all_reduce_population_count(m)
        return _sb

    def split(M, tp):
        def sb(j, boff):
            base = j * L
            valid = (base + lanes) < M
            iv = idxs_v[pl.ds(base, L)]
            kv = plsc.load_gather(keys_v, [iv], mask=valid)
            m = (kv >= tp) & valid
            c = plsc.cumsum(m.astype(jnp.int32))
            plsc.store_scatter(idxs_v, [boff + c - 1], iv, mask=m)
            return boff + plsc.all_reduce_population_count(m)
        return lax.fori_loop(0, (M + L - 1) // L, sb, zeros)

    def chunk_copy(row, c):
        return pltpu.make_async_copy(
            x_hbm.at[row, pl.ds(c * CWORDS, CWORDS)],
            row_v.at[pl.ds(c * CWORDS, CWORDS)], sems[c])

    def row_body(r, _):
        row = row0 + r
        for c in range(NCHUNK):
            chunk_copy(row, c).start()
        zero_hist(NHIST)
        for c in range(NCHUNK):
            chunk_copy(row, c).wait()
            pass1_chunk(c * CWORDS)

        b1 = find_bin(NHIST)
        tp = ((b1 ^ 0x80) & 0xFF) << 24
        boff = split1(tp)
        M = jnp.max(boff)

        for shift in (16, 8, 0):
            hi = tp | ((1 << (shift + 8)) - 1)
            hist_pass(M, shift, hi)
            b = find_bin(1)
            tp = tp | (b << shift)
            boff = split(M, tp)
            M = jnp.max(boff)

        def fb(j, carry):
            doff, boff = carry
            base = j * L
            valid = (base + lanes) < M
            iv = idxs_v[pl.ds(base, L)]
            kv = plsc.load_gather(keys_v, [iv], mask=valid)
            m_gt = (kv > tp) & valid
            m_eq = (kv == tp) & valid
            c_gt = plsc.cumsum(m_gt.astype(jnp.int32))
            plsc.store_scatter(didx_v, [doff + c_gt - 1], iv, mask=m_gt)
            c_eq = plsc.cumsum(m_eq.astype(jnp.int32))
            plsc.store_scatter(idxs_v, [boff + c_eq - 1], iv, mask=m_eq)
            doff = doff + plsc.all_reduce_population_count(m_gt)
            boff = boff + plsc.all_reduce_population_count(m_eq)
            return doff, boff
        doff, _ = lax.fori_loop(0, (M + L - 1) // L, fb, (zeros, zeros))
        kneed = K - jnp.max(doff)
        doffs = jnp.max(doff)

        def ap(j, _):
            pos = j * L + lanes
            m = pos < kneed
            iv = idxs_v[pl.ds(j * L, L)]
            plsc.store_scatter(didx_v, [doffs + pos], iv, mask=m)
            return 0
        lax.fori_loop(0, (kneed + L - 1) // L, ap, 0)

        Is = [didx_v[pl.ds(a * L, L)] for a in range(K // L)]
        Ks = [plsc.load_gather(keys_v, [iv]) for iv in Is]
        rsplat = jnp.full((L,), r, jnp.int32)

        @plsc.parallel_loop(0, K, unroll=2)
        def _rk(c):
            csplat = jnp.full((L,), c, jnp.int32)
            ic = plsc.load_gather(didx_v, [csplat])
            kc = plsc.load_gather(keys_v, [ic])
            rank = zeros
            for a in range(K // L):
                m = (Ks[a] > kc) | ((Ks[a] == kc) & (Is[a] < ic))
                rank = rank + plsc.all_reduce_population_count(m)
            plsc.store_scatter(ostage_v, [rsplat, rank], ic, mask=lane0)
        return 0

    lax.fori_loop(0, ROWS_PER_W, row_body, 0)
    pltpu.make_async_copy(ostage_v, out_hbm.at[pl.ds(row0, ROWS_PER_W)],
                          sem_out).start()
    pltpu.make_async_copy(ostage_v, out_hbm.at[pl.ds(row0, ROWS_PER_W)],
                          sem_out).wait()


@jax.jit
def kernel(input_tensor):
    x_i32 = lax.bitcast_convert_type(input_tensor, jnp.int32)
    mesh = plsc.VectorSubcoreMesh(core_axis_name="c", subcore_axis_name="s",
                                  num_cores=NC, num_subcores=NS)
    f = pl.kernel(
        _body,
        out_type=jax.ShapeDtypeStruct((NROWS, K), jnp.int32),
        mesh=mesh,
        scratch_types=[
            pltpu.VMEM((NCOLS,), jnp.int32),
            pltpu.VMEM((NCOLS,), jnp.int32),
            pltpu.VMEM((NCOLS,), jnp.int32),
            pltpu.VMEM((HSIZE * NHIST,), jnp.int32),
            pltpu.VMEM((K,), jnp.int32),
            pltpu.VMEM((ROWS_PER_W, K), jnp.int32),
            [pltpu.SemaphoreType.DMA] * NCHUNK,
            pltpu.SemaphoreType.DMA,
        ],
        compiler_params=pltpu.CompilerParams(
            needs_layout_passes=False,
            use_tc_tiling_on_sc=False,
        ),
    )
    return f(x_i32)


# speedup vs baseline: 3.0375x; 1.2202x over previous
import jax
import jax.numpy as jnp
from jax import lax
from jax.experimental import pallas as pl
from jax.experimental.pallas import tpu as pltpu
from jax.experimental.pallas import tpu_sc as plsc

NROWS = 128
NCOLS = 32768
K = 64
NC, NS, L = 2, 16, 16
NW = NC * NS
ROWS_PER_W = NROWS // NW
NVREG = NCOLS // L
NBINS = 256
HSTRIDE = NBINS + 1
HSIZE = HSTRIDE * L
NHIST = 4
NCHUNK = 4
CWORDS = NCOLS // NCHUNK
CVREG = CWORDS // L


def _body(x_hbm, out_hbm, row_v, keys_v, idxs_v, hist_v, didx_v,
          ostage_v, sems, sem_out):
    wid = lax.axis_index("s") * NC + lax.axis_index("c")
    row0 = wid * ROWS_PER_W
    lanes = lax.iota(jnp.int32, L)
    ones = jnp.ones((L,), jnp.int32)
    zeros = jnp.zeros((L,), jnp.int32)
    tmask = lanes >= 0
    lane0 = lanes == 0
    lanebins = lanes * HSTRIDE

    def zero_hist(ncopies):
        @plsc.parallel_loop(0, (HSTRIDE * ncopies * L) // L, unroll=4)
        def _zb(i):
            hist_v[pl.ds(i * L, L)] = zeros

    def pass1_chunk(c0):
        @plsc.parallel_loop(0, CVREG // NHIST, unroll=2)
        def _p1(j):
            for t in range(NHIST):
                base = c0 + (j * NHIST + t) * L
                k = row_v[pl.ds(base, L)]
                ikey = jnp.where(k < 0, k ^ jnp.int32(0x7FFFFFFF), k)
                keys_v[pl.ds(base, L)] = ikey
                digit = ((ikey >> 24) & 0xFF) ^ 0x80
                plsc.addupdate_scatter(
                    hist_v, [digit + (lanebins + t * HSIZE)], ones,
                    mask=tmask)

    def hist_pass(M, shift, hi):
        zero_hist(1)
        def hb(j, _):
            base = j * L
            valid = (base + lanes) < M
            iv = idxs_v[pl.ds(base, L)]
            kv = plsc.load_gather(keys_v, [iv], mask=valid)
            digit = jnp.where(kv > hi, 255, (kv >> shift) & 0xFF)
            plsc.addupdate_scatter(hist_v, [digit + lanebins], ones,
                                   mask=valid)
            return 0
        lax.fori_loop(0, (M + L - 1) // L, hb, 0)

    def find_bin(ncopies):
        b = jnp.int32(-1)
        found = jnp.bool_(False)
        sab = jnp.int32(0)
        for v in range(L - 1, -1, -1):
            def lr(l, acc):
                a = acc
                for c in range(ncopies):
                    a = a + hist_v[pl.ds(l * HSTRIDE + c * HSIZE + v * L, L)]
                return a
            t_v = lax.fori_loop(0, L, lr, zeros, unroll=4)
            rc_v = lax.rev(plsc.cumsum(lax.rev(t_v, (0,))), (0,))
            need = K - sab
            cnt = jnp.sum((rc_v >= need).astype(jnp.int32))
            hit = cnt > 0
            b = jnp.where(found | (~hit), b, v * L + cnt - 1)
            found = found | hit
            sab = sab + jnp.max(rc_v)
        return b

    def split1(tp):
        @plsc.parallel_loop(0, NVREG, unroll=4, carry=zeros)
        def _sb(j, boff):
            base = j * L
            kv = keys_v[pl.ds(base, L)]
            m = kv >= tp
            c = plsc.cumsum(m.astype(jnp.int32))
            plsc.store_scatter(idxs_v, [boff + c - 1], base + lanes, mask=m)
            return boff + plsc.all_reduce_population_count(m)
        return _sb

    def split(M, tp):
        def sb(j, boff):
            base = j * L
            valid = (base + lanes) < M
            iv = idxs_v[pl.ds(base, L)]
            kv = plsc.load_gather(keys_v, [iv], mask=valid)
            m = (kv >= tp) & valid
            c = plsc.cumsum(m.astype(jnp.int32))
            plsc.store_scatter(idxs_v, [boff + c - 1], iv, mask=m)
            return boff + plsc.all_reduce_population_count(m)
        return lax.fori_loop(0, (M + L - 1) // L, sb, zeros)

    def chunk_copy(row, c):
        return pltpu.make_async_copy(
            x_hbm.at[row, pl.ds(c * CWORDS, CWORDS)],
            row_v.at[pl.ds(c * CWORDS, CWORDS)], sems[c])

    def row_body(r, _):
        row = row0 + r
        for c in range(NCHUNK):
            chunk_copy(row, c).start()
        zero_hist(NHIST)
        for c in range(NCHUNK):
            chunk_copy(row, c).wait()
            pass1_chunk(c * CWORDS)

        b1 = find_bin(NHIST)
        tp = ((b1 ^ 0x80) & 0xFF) << 24
        boff = split1(tp)
        M = jnp.max(boff)

        for shift in (16, 8, 0):
            hi = tp | ((1 << (shift + 8)) - 1)
            hist_pass(M, shift, hi)
            b = find_bin(1)
            tp = tp | (b << shift)
            boff = split(M, tp)
            M = jnp.max(boff)

        def fb(j, carry):
            doff, boff = carry
            base = j * L
            valid = (base + lanes) < M
            iv = idxs_v[pl.ds(base, L)]
            kv = plsc.load_gather(keys_v, [iv], mask=valid)
            m_gt = (kv > tp) & valid
            m_eq = (kv == tp) & valid
            c_gt = plsc.cumsum(m_gt.astype(jnp.int32))
            plsc.store_scatter(didx_v, [doff + c_gt - 1], iv, mask=m_gt)
            c_eq = plsc.cumsum(m_eq.astype(jnp.int32))
            plsc.store_scatter(idxs_v, [boff + c_eq - 1], iv, mask=m_eq)
            doff = doff + plsc.all_reduce_population_count(m_gt)
            boff = boff + plsc.all_reduce_population_count(m_eq)
            return doff, boff
        doff, _ = lax.fori_loop(0, (M + L - 1) // L, fb, (zeros, zeros))
        kneed = K - jnp.max(doff)
        doffs = jnp.max(doff)

        def ap(j, _):
            pos = j * L + lanes
            m = pos < kneed
            iv = idxs_v[pl.ds(j * L, L)]
            plsc.store_scatter(didx_v, [doffs + pos], iv, mask=m)
            return 0
        lax.fori_loop(0, (kneed + L - 1) // L, ap, 0)

        Is = [didx_v[pl.ds(a * L, L)] for a in range(K // L)]
        Ks = [plsc.load_gather(keys_v, [iv]) for iv in Is]
        rsplat = jnp.full((L,), r, jnp.int32)

        @plsc.parallel_loop(0, K, unroll=2)
        def _rk(c):
            csplat = jnp.full((L,), c, jnp.int32)
            ic = plsc.load_gather(didx_v, [csplat])
            kc = plsc.load_gather(keys_v, [ic])
            rank = zeros
            for a in range(K // L):
                m = (Ks[a] > kc) | ((Ks[a] == kc) & (Is[a] < ic))
                rank = rank + plsc.all_reduce_population_count(m)
            plsc.store_scatter(ostage_v, [rsplat, rank], ic, mask=lane0)
        return 0

    lax.fori_loop(0, ROWS_PER_W, row_body, 0)
    pltpu.make_async_copy(ostage_v, out_hbm.at[pl.ds(row0, ROWS_PER_W)],
                          sem_out).start()
    pltpu.make_async_copy(ostage_v, out_hbm.at[pl.ds(row0, ROWS_PER_W)],
                          sem_out).wait()


@jax.jit
def kernel(input_tensor):
    x_i32 = lax.bitcast_convert_type(input_tensor, jnp.int32)
    mesh = plsc.VectorSubcoreMesh(core_axis_name="c", subcore_axis_name="s",
                                  num_cores=NC, num_subcores=NS)
    f = pl.kernel(
        _body,
        out_type=jax.ShapeDtypeStruct((NROWS, K), jnp.int32),
        mesh=mesh,
        scratch_types=[
            pltpu.VMEM((NCOLS,), jnp.int32),
            pltpu.VMEM((NCOLS,), jnp.int32),
            pltpu.VMEM((NCOLS,), jnp.int32),
            pltpu.VMEM((HSIZE * NHIST,), jnp.int32),
            pltpu.VMEM((K,), jnp.int32),
            pltpu.VMEM((ROWS_PER_W, K), jnp.int32),
            [pltpu.SemaphoreType.DMA] * NCHUNK,
            pltpu.SemaphoreType.DMA,
        ],
        compiler_params=pltpu.CompilerParams(
            needs_layout_passes=False,
            use_tc_tiling_on_sc=True,
        ),
    )
    return f(x_i32)


# speedup vs baseline: 3.3706x; 1.1097x over previous
import jax
import jax.numpy as jnp
from jax import lax
from jax.experimental import pallas as pl
from jax.experimental.pallas import tpu as pltpu
from jax.experimental.pallas import tpu_sc as plsc

NROWS = 128
NCOLS = 32768
K = 64
NC, NS, L = 2, 16, 16
NW = NC * NS
ROWS_PER_W = NROWS // NW
NVREG = NCOLS // L
NBINS = 256
HSTRIDE = NBINS + 1
HSIZE = HSTRIDE * L
NHIST = 4
NCHUNK = 4
CWORDS = NCOLS // NCHUNK
CVREG = CWORDS // L


def _tr(k):
    return jnp.where(k < 0, k ^ jnp.int32(0x7FFFFFFF), k)


def _body(x_hbm, out_hbm, row_v, idxs_v, hist_v, didx_v, ostage_v, sems,
          sem_out):
    wid = lax.axis_index("s") * NC + lax.axis_index("c")
    row0 = wid * ROWS_PER_W
    lanes = lax.iota(jnp.int32, L)
    ones = jnp.ones((L,), jnp.int32)
    zeros = jnp.zeros((L,), jnp.int32)
    tmask = lanes >= 0
    lane0 = lanes == 0
    lanebins = lanes * HSTRIDE

    def zero_hist(ncopies):
        @plsc.parallel_loop(0, (HSTRIDE * ncopies * L) // L, unroll=4)
        def _zb(i):
            hist_v[pl.ds(i * L, L)] = zeros

    def pass1_chunk(c0):
        @plsc.parallel_loop(0, CVREG // NHIST, unroll=2)
        def _p1(j):
            for t in range(NHIST):
                base = c0 + (j * NHIST + t) * L
                ikey = _tr(row_v[pl.ds(base, L)])
                digit = ((ikey >> 24) & 0xFF) ^ 0x80
                plsc.addupdate_scatter(
                    hist_v, [digit + (lanebins + t * HSIZE)], ones,
                    mask=tmask)

    def gather_keys(iv, valid):
        return _tr(plsc.load_gather(row_v, [iv], mask=valid))

    def hist_pass(M, shift, hi):
        zero_hist(1)
        def hb(j, _):
            base = j * L
            valid = (base + lanes) < M
            kv = gather_keys(idxs_v[pl.ds(base, L)], valid)
            digit = jnp.where(kv > hi, 255, (kv >> shift) & 0xFF)
            plsc.addupdate_scatter(hist_v, [digit + lanebins], ones,
                                   mask=valid)
            return 0
        lax.fori_loop(0, (M + L - 1) // L, hb, 0)

    def find_bin(ncopies):
        def fbv(i, carry):
            b, found, sab = carry
            v = L - 1 - i
            def lr(l, acc):
                a = acc
                for c in range(ncopies):
                    a = a + hist_v[pl.ds(l * HSTRIDE + c * HSIZE + v * L, L)]
                return a
            t_v = lax.fori_loop(0, L, lr, zeros, unroll=4)
            rc_v = lax.rev(plsc.cumsum(lax.rev(t_v, (0,))), (0,))
            need = K - sab
            cnt = jnp.sum((rc_v >= need).astype(jnp.int32))
            hit = cnt > 0
            b = jnp.where(found | (~hit), b, v * L + cnt - 1)
            return b, found | hit, sab + jnp.max(rc_v)
        b, _, _ = lax.fori_loop(
            0, L, fbv, (jnp.int32(-1), jnp.bool_(False), jnp.int32(0)))
        return b

    def split1(tp):
        @plsc.parallel_loop(0, NVREG, unroll=4, carry=zeros)
        def _sb(j, boff):
            base = j * L
            kv = _tr(row_v[pl.ds(base, L)])
            m = kv >= tp
            c = plsc.cumsum(m.astype(jnp.int32))
            plsc.store_scatter(idxs_v, [boff + c - 1], base + lanes, mask=m)
            return boff + plsc.all_reduce_population_count(m)
        return _sb

    def split(M, tp):
        def sb(j, boff):
            base = j * L
            valid = (base + lanes) < M
            iv = idxs_v[pl.ds(base, L)]
            kv = gather_keys(iv, valid)
            m = (kv >= tp) & valid
            c = plsc.cumsum(m.astype(jnp.int32))
            plsc.store_scatter(idxs_v, [boff + c - 1], iv, mask=m)
            return boff + plsc.all_reduce_population_count(m)
        return lax.fori_loop(0, (M + L - 1) // L, sb, zeros)

    def chunk_copy(row, c):
        return pltpu.make_async_copy(
            x_hbm.at[row, pl.ds(c * CWORDS, CWORDS)],
            row_v.at[pl.ds(c * CWORDS, CWORDS)], sems[c])

    def row_body(r, _):
        row = row0 + r
        for c in range(NCHUNK):
            chunk_copy(row, c).start()
        zero_hist(NHIST)
        for c in range(NCHUNK):
            chunk_copy(row, c).wait()
            pass1_chunk(c * CWORDS)

        b1 = find_bin(NHIST)
        tp = ((b1 ^ 0x80) & 0xFF) << 24
        boff = split1(tp)

        def level(lvl, carry):
            tp, M = carry
            shift = 16 - lvl * 8
            hi = tp | (lax.shift_left(jnp.int32(1), shift + 8) - 1)
            hist_pass(M, shift, hi)
            b = find_bin(1)
            tp = tp | lax.shift_left(b, shift)
            boff = split(M, tp)
            return tp, jnp.max(boff)
        tp, M = lax.fori_loop(0, 3, level, (tp, jnp.max(boff)))

        def fb(j, carry):
            doff, boff = carry
            base = j * L
            valid = (base + lanes) < M
            iv = idxs_v[pl.ds(base, L)]
            kv = gather_keys(iv, valid)
            m_gt = (kv > tp) & valid
            m_eq = (kv == tp) & valid
            c_gt = plsc.cumsum(m_gt.astype(jnp.int32))
            plsc.store_scatter(didx_v, [doff + c_gt - 1], iv, mask=m_gt)
            c_eq = plsc.cumsum(m_eq.astype(jnp.int32))
            plsc.store_scatter(idxs_v, [boff + c_eq - 1], iv, mask=m_eq)
            doff = doff + plsc.all_reduce_population_count(m_gt)
            boff = boff + plsc.all_reduce_population_count(m_eq)
            return doff, boff
        doff, _ = lax.fori_loop(0, (M + L - 1) // L, fb, (zeros, zeros))
        kneed = K - jnp.max(doff)
        doffs = jnp.max(doff)

        def ap(j, _):
            pos = j * L + lanes
            m = pos < kneed
            iv = idxs_v[pl.ds(j * L, L)]
            plsc.store_scatter(didx_v, [doffs + pos], iv, mask=m)
            return 0
        lax.fori_loop(0, (kneed + L - 1) // L, ap, 0)

        Is = [didx_v[pl.ds(a * L, L)] for a in range(K // L)]
        Ks = [gather_keys(iv, tmask) for iv in Is]
        rsplat = jnp.full((L,), r, jnp.int32)

        @plsc.parallel_loop(0, K, unroll=2)
        def _rk(c):
            csplat = jnp.full((L,), c, jnp.int32)
            ic = plsc.load_gather(didx_v, [csplat])
            kc = gather_keys(ic, tmask)
            rank = zeros
            for a in range(K // L):
                m = (Ks[a] > kc) | ((Ks[a] == kc) & (Is[a] < ic))
                rank = rank + plsc.all_reduce_population_count(m)
            plsc.store_scatter(ostage_v, [rsplat, rank], ic, mask=lane0)
        return 0

    lax.fori_loop(0, ROWS_PER_W, row_body, 0)
    pltpu.make_async_copy(ostage_v, out_hbm.at[pl.ds(row0, ROWS_PER_W)],
                          sem_out).start()
    pltpu.make_async_copy(ostage_v, out_hbm.at[pl.ds(row0, ROWS_PER_W)],
                          sem_out).wait()


@jax.jit
def kernel(input_tensor):
    x_i32 = lax.bitcast_convert_type(input_tensor, jnp.int32)
    mesh = plsc.VectorSubcoreMesh(core_axis_name="c", subcore_axis_name="s",
                                  num_cores=NC, num_subcores=NS)
    f = pl.kernel(
        _body,
        out_type=jax.ShapeDtypeStruct((NROWS, K), jnp.int32),
        mesh=mesh,
        scratch_types=[
            pltpu.VMEM((NCOLS,), jnp.int32),
            pltpu.VMEM((NCOLS,), jnp.int32),
            pltpu.VMEM((HSIZE * NHIST,), jnp.int32),
            pltpu.VMEM((K,), jnp.int32),
            pltpu.VMEM((ROWS_PER_W, K), jnp.int32),
            [pltpu.SemaphoreType.DMA] * NCHUNK,
            pltpu.SemaphoreType.DMA,
        ],
        compiler_params=pltpu.CompilerParams(
            needs_layout_passes=False,
            use_tc_tiling_on_sc=True,
        ),
    )
    return f(x_i32)


# speedup vs baseline: 3.5238x; 1.0455x over previous
import jax
import jax.numpy as jnp
from jax import lax
from jax.experimental import pallas as pl
from jax.experimental.pallas import tpu as pltpu
from jax.experimental.pallas import tpu_sc as plsc

NROWS = 128
NCOLS = 32768
K = 64
NC, NS, L = 2, 16, 16
NW = NC * NS
ROWS_PER_W = NROWS // NW
NVREG = NCOLS // L
NBINS = 256
HSTRIDE = NBINS + 1
HSIZE = HSTRIDE * L
NHIST = 4
NCHUNK = 4
CWORDS = NCOLS // NCHUNK
CVREG = CWORDS // L


def _tr(k):
    return jnp.where(k < 0, k ^ jnp.int32(0x7FFFFFFF), k)


def _body(x_hbm, out_hbm, row_v, idxs_v, hist_v, didx_v, ostage_v, sems,
          sem_out):
    wid = lax.axis_index("s") * NC + lax.axis_index("c")
    row0 = wid * ROWS_PER_W
    lanes = lax.iota(jnp.int32, L)
    ones = jnp.ones((L,), jnp.int32)
    zeros = jnp.zeros((L,), jnp.int32)
    tmask = lanes >= 0
    lane0 = lanes == 0
    lanebins = lanes * HSTRIDE

    def zero_hist(ncopies):
        @plsc.parallel_loop(0, (HSTRIDE * ncopies * L) // L, unroll=4)
        def _zb(i):
            hist_v[pl.ds(i * L, L)] = zeros

    def pass1_chunk(c0):
        @plsc.parallel_loop(0, CVREG // NHIST, unroll=4)
        def _p1(j):
            for t in range(NHIST):
                base = c0 + (j * NHIST + t) * L
                ikey = _tr(row_v[pl.ds(base, L)])
                digit = ((ikey >> 24) & 0xFF) ^ 0x80
                plsc.addupdate_scatter(
                    hist_v, [digit + (lanebins + t * HSIZE)], ones,
                    mask=tmask)

    def gather_keys(iv, valid):
        return _tr(plsc.load_gather(row_v, [iv], mask=valid))

    def hist_pass(M, shift, hi):
        zero_hist(1)
        def hb(j, _):
            base = j * L
            valid = (base + lanes) < M
            kv = gather_keys(idxs_v[pl.ds(base, L)], valid)
            digit = jnp.where(kv > hi, 255, (kv >> shift) & 0xFF)
            plsc.addupdate_scatter(hist_v, [digit + lanebins], ones,
                                   mask=valid)
            return 0
        lax.fori_loop(0, (M + L - 1) // L, hb, 0)

    def find_bin(ncopies):
        def fbv(i, carry):
            b, found, sab = carry
            v = L - 1 - i
            def lr(l, acc):
                a = acc
                for c in range(ncopies):
                    a = a + hist_v[pl.ds(l * HSTRIDE + c * HSIZE + v * L, L)]
                return a
            t_v = lax.fori_loop(0, L, lr, zeros, unroll=4)
            rc_v = lax.rev(plsc.cumsum(lax.rev(t_v, (0,))), (0,))
            need = K - sab
            cnt = jnp.sum((rc_v >= need).astype(jnp.int32))
            hit = cnt > 0
            b = jnp.where(found | (~hit), b, v * L + cnt - 1)
            return b, found | hit, sab + jnp.max(rc_v)
        b, _, _ = lax.fori_loop(
            0, L, fbv, (jnp.int32(-1), jnp.bool_(False), jnp.int32(0)))
        return b

    def split1(tp):
        @plsc.parallel_loop(0, NVREG, unroll=8, carry=zeros)
        def _sb(j, boff):
            base = j * L
            kv = _tr(row_v[pl.ds(base, L)])
            m = kv >= tp
            c = plsc.cumsum(m.astype(jnp.int32))
            plsc.store_scatter(idxs_v, [boff + c - 1], base + lanes, mask=m)
            return boff + plsc.all_reduce_population_count(m)
        return _sb

    def split(M, tp):
        def sb(j, boff):
            base = j * L
            valid = (base + lanes) < M
            iv = idxs_v[pl.ds(base, L)]
            kv = gather_keys(iv, valid)
            m = (kv >= tp) & valid
            c = plsc.cumsum(m.astype(jnp.int32))
            plsc.store_scatter(idxs_v, [boff + c - 1], iv, mask=m)
            return boff + plsc.all_reduce_population_count(m)
        return lax.fori_loop(0, (M + L - 1) // L, sb, zeros)

    def chunk_copy(row, c):
        return pltpu.make_async_copy(
            x_hbm.at[row, pl.ds(c * CWORDS, CWORDS)],
            row_v.at[pl.ds(c * CWORDS, CWORDS)], sems[c])

    def row_body(r, _):
        row = row0 + r
        for c in range(NCHUNK):
            chunk_copy(row, c).start()
        zero_hist(NHIST)
        for c in range(NCHUNK):
            chunk_copy(row, c).wait()
            pass1_chunk(c * CWORDS)

        b1 = find_bin(NHIST)
        tp = ((b1 ^ 0x80) & 0xFF) << 24
        boff = split1(tp)

        def level(lvl, carry):
            tp, M = carry
            shift = 16 - lvl * 8
            hi = tp | (lax.shift_left(jnp.int32(1), shift + 8) - 1)
            hist_pass(M, shift, hi)
            b = find_bin(1)
            tp = tp | lax.shift_left(b, shift)
            boff = split(M, tp)
            return tp, jnp.max(boff)
        tp, M = lax.fori_loop(0, 3, level, (tp, jnp.max(boff)))

        def fb(j, carry):
            doff, boff = carry
            base = j * L
            valid = (base + lanes) < M
            iv = idxs_v[pl.ds(base, L)]
            kv = gather_keys(iv, valid)
            m_gt = (kv > tp) & valid
            m_eq = (kv == tp) & valid
            c_gt = plsc.cumsum(m_gt.astype(jnp.int32))
            plsc.store_scatter(didx_v, [doff + c_gt - 1], iv, mask=m_gt)
            c_eq = plsc.cumsum(m_eq.astype(jnp.int32))
            plsc.store_scatter(idxs_v, [boff + c_eq - 1], iv, mask=m_eq)
            doff = doff + plsc.all_reduce_population_count(m_gt)
            boff = boff + plsc.all_reduce_population_count(m_eq)
            return doff, boff
        doff, _ = lax.fori_loop(0, (M + L - 1) // L, fb, (zeros, zeros))
        kneed = K - jnp.max(doff)
        doffs = jnp.max(doff)

        def ap(j, _):
            pos = j * L + lanes
            m = pos < kneed
            iv = idxs_v[pl.ds(j * L, L)]
            plsc.store_scatter(didx_v, [doffs + pos], iv, mask=m)
            return 0
        lax.fori_loop(0, (kneed + L - 1) // L, ap, 0)

        Is = [didx_v[pl.ds(a * L, L)] for a in range(K // L)]
        Ks = [gather_keys(iv, tmask) for iv in Is]
        rsplat = jnp.full((L,), r, jnp.int32)

        @plsc.parallel_loop(0, K, unroll=2)
        def _rk(c):
            csplat = jnp.full((L,), c, jnp.int32)
            ic = plsc.load_gather(didx_v, [csplat])
            kc = gather_keys(ic, tmask)
            rank = zeros
            for a in range(K // L):
                m = (Ks[a] > kc) | ((Ks[a] == kc) & (Is[a] < ic))
                rank = rank + plsc.all_reduce_population_count(m)
            plsc.store_scatter(ostage_v, [rsplat, rank], ic, mask=lane0)
        return 0

    lax.fori_loop(0, ROWS_PER_W, row_body, 0)
    pltpu.make_async_copy(ostage_v, out_hbm.at[pl.ds(row0, ROWS_PER_W)],
                          sem_out).start()
    pltpu.make_async_copy(ostage_v, out_hbm.at[pl.ds(row0, ROWS_PER_W)],
                          sem_out).wait()


@jax.jit
def kernel(input_tensor):
    x_i32 = lax.bitcast_convert_type(input_tensor, jnp.int32)
    mesh = plsc.VectorSubcoreMesh(core_axis_name="c", subcore_axis_name="s",
                                  num_cores=NC, num_subcores=NS)
    f = pl.kernel(
        _body,
        out_type=jax.ShapeDtypeStruct((NROWS, K), jnp.int32),
        mesh=mesh,
        scratch_types=[
            pltpu.VMEM((NCOLS,), jnp.int32),
            pltpu.VMEM((NCOLS,), jnp.int32),
            pltpu.VMEM((HSIZE * NHIST,), jnp.int32),
            pltpu.VMEM((K,), jnp.int32),
            pltpu.VMEM((ROWS_PER_W, K), jnp.int32),
            [pltpu.SemaphoreType.DMA] * NCHUNK,
            pltpu.SemaphoreType.DMA,
        ],
        compiler_params=pltpu.CompilerParams(
            needs_layout_passes=False,
            use_tc_tiling_on_sc=True,
        ),
    )
    return f(x_i32)
